# baseline (speedup 1.0000x reference)
import jax
import jax.numpy as jnp
from jax import lax
from jax.experimental import pallas as pl
from jax.experimental.pallas import tpu as pltpu
from jax.experimental.pallas import tpu_sc as plsc

N = 10000
E = 320000
D = 128
ED = 16
NC = 2
NS = 16
NW = NC * NS
CH = 128
NCHUNK = E // CH
NK = 80
EPAD = NW * NK * CH
NB = 1000
EB = 4000
ZR = 128
NP = 10240

_f32 = jnp.float32


def _tcB_body(xi_ref, xj_ref, ef_ref,
              ew1a_ref, ew1b_ref, ew1c_ref, eb1_ref, ew2_ref, eb2_ref,
              nw1a_ref, nw1b_ref, nw1c_ref, nb1_ref,
              ef2_ref, u_ref):
    xi = xi_ref[...]
    xj = xj_ref[...]
    ef = ef_ref[...]
    pe = (jnp.dot(xi, ew1a_ref[...], preferred_element_type=_f32)
          + jnp.dot(xj, ew1b_ref[...], preferred_element_type=_f32)
          + jnp.dot(ef, ew1c_ref[...], preferred_element_type=_f32)
          + eb1_ref[...])
    t = jnp.maximum(pe, 0.0)
    ef2 = jnp.dot(t, ew2_ref[...], preferred_element_type=_f32) + eb2_ref[...]
    ef2_ref[...] = ef2
    pn = (jnp.dot(xi, nw1a_ref[...], preferred_element_type=_f32)
          + jnp.dot(xj, nw1b_ref[...], preferred_element_type=_f32)
          + jnp.dot(ef2, nw1c_ref[...], preferred_element_type=_f32)
          + nb1_ref[...])
    u_ref[...] = jnp.maximum(pn, 0.0)


def _tcB(xi, xj, ef, eW1a, eW1b, eW1c, eb1, eW2, eb2, nW1a, nW1b, nW1c, nb1):
    grid = E // EB
    full = lambda shape: pl.BlockSpec(shape, lambda i: tuple(0 for _ in shape))
    return pl.pallas_call(
        _tcB_body,
        grid=(grid,),
        in_specs=[
            pl.BlockSpec((EB, D), lambda i: (i, 0)),
            pl.BlockSpec((EB, D), lambda i: (i, 0)),
            pl.BlockSpec((EB, ED), lambda i: (i, 0)),
            full((D, 32)), full((D, 32)), full((ED, 32)), full((1, 32)),
            full((32, ED)), full((1, ED)),
            full((D, D)), full((D, D)), full((ED, D)), full((1, D)),
        ],
        out_specs=[
            pl.BlockSpec((EB, ED), lambda i: (i, 0)),
            pl.BlockSpec((EB, D), lambda i: (i, 0)),
        ],
        out_shape=[
            jax.ShapeDtypeStruct((E, ED), _f32),
            jax.ShapeDtypeStruct((EPAD, D), _f32),
        ],
    )(xi, xj, ef, eW1a, eW1b, eW1c, eb1, eW2, eb2, nW1a, nW1b, nW1c, nb1)


def _tcC_body(agg_ref, deg_ref, w_ref, b_ref, h_ref):
    a = agg_ref[0] + agg_ref[1]
    d = deg_ref[0, :, 0:1] + deg_ref[1, :, 0:1]
    h_ref[...] = jnp.dot(a, w_ref[...], preferred_element_type=_f32) + d * b_ref[...]


def _tcC(agg, deg, nW2, nb2):
    grid = N // NB
    return pl.pallas_call(
        _tcC_body,
        grid=(grid,),
        in_specs=[
            pl.BlockSpec((2, NB, D), lambda i: (0, i, 0)),
            pl.BlockSpec((2, NB, ED), lambda i: (0, i, 0)),
            pl.BlockSpec((D, D), lambda i: (0, 0)),
            pl.BlockSpec((1, D), lambda i: (0, 0)),
        ],
        out_specs=pl.BlockSpec((NB, D), lambda i: (i, 0)),
        out_shape=jax.ShapeDtypeStruct((N, D), _f32),
    )(agg, deg, nW2, nb2)


def _mesh():
    return plsc.VectorSubcoreMesh(core_axis_name="c", subcore_axis_name="s",
                                  num_cores=NC, num_subcores=NS)


def _sc_gather_body(h_hbm, dst_hbm, src_hbm, xi_hbm, xj_hbm,
                    idxd_all, idxs_all, xi0, xi1, xj0, xj1,
                    gd0, gd1, gs0, gs1, wd0, wd1, ws0, ws1):
    cid = lax.axis_index("c")
    sid = lax.axis_index("s")
    wid = sid * NC + cid
    base = wid * (NK * CH)
    pltpu.sync_copy(dst_hbm.at[pl.ds(base, NK * CH)], idxd_all)
    pltpu.sync_copy(src_hbm.at[pl.ds(base, NK * CH)], idxs_all)

    def pair(m, carry):
        k0 = 2 * m
        off0 = base + k0 * CH
        off1 = off0 + CH
        cd0 = pltpu.async_copy(h_hbm.at[idxd_all.at[pl.ds(k0 * CH, CH)]], xi0, gd0)
        cs0 = pltpu.async_copy(h_hbm.at[idxs_all.at[pl.ds(k0 * CH, CH)]], xj0, gs0)
        cd1 = pltpu.async_copy(h_hbm.at[idxd_all.at[pl.ds(k0 * CH + CH, CH)]], xi1, gd1)
        cs1 = pltpu.async_copy(h_hbm.at[idxs_all.at[pl.ds(k0 * CH + CH, CH)]], xj1, gs1)
        cd0.wait()
        w0 = pltpu.async_copy(xi0, xi_hbm.at[pl.ds(off0, CH)], wd0)
        cs0.wait()
        w1 = pltpu.async_copy(xj0, xj_hbm.at[pl.ds(off0, CH)], ws0)
        cd1.wait()
        w2 = pltpu.async_copy(xi1, xi_hbm.at[pl.ds(off1, CH)], wd1)
        cs1.wait()
        w3 = pltpu.async_copy(xj1, xj_hbm.at[pl.ds(off1, CH)], ws1)
        w0.wait()
        w1.wait()
        w2.wait()
        w3.wait()
        return carry

    lax.fori_loop(0, NK // 2, pair, None)


def _sc_gather(h, dst, src):
    fn = pl.kernel(
        _sc_gather_body,
        out_type=[
            jax.ShapeDtypeStruct((EPAD, D), _f32),
            jax.ShapeDtypeStruct((EPAD, D), _f32),
        ],
        mesh=_mesh(),
        scratch_types=[
            pltpu.VMEM((NK * CH,), jnp.int32),
            pltpu.VMEM((NK * CH,), jnp.int32),
            pltpu.VMEM((CH, D), _f32),
            pltpu.VMEM((CH, D), _f32),
            pltpu.VMEM((CH, D), _f32),
            pltpu.VMEM((CH, D), _f32),
        ] + [pltpu.SemaphoreType.DMA] * 8,
    )
    return fn(h, dst, src)


def _sc_scatter_body(u_hbm, dst_hbm, out_hbm, idx0, idx1, pay0, pay1, acc_sh,
                     is0, is1, ps0, ps1):
    cid = lax.axis_index("c")
    sid = lax.axis_index("s")
    wid = sid * NC + cid

    zvec = jnp.zeros((16,), _f32)

    def zbody(i, carry):
        r = i // (D // 16)
        c = (i % (D // 16)) * 16
        pay0[r, pl.ds(c, 16)] = zvec
        return carry

    lax.fori_loop(0, CH * (D // 16), zbody, None)
    for j in range(5):
        pltpu.sync_copy(pay0, acc_sh.at[pl.ds(sid * 640 + j * CH, CH)])
    plsc.subcore_barrier()

    base = wid * (NK * CH)

    def pair(m, carry):
        off0 = base + (2 * m) * CH
        off1 = off0 + CH
        i0 = pltpu.async_copy(dst_hbm.at[pl.ds(off0, CH)], idx0, is0)
        p0 = pltpu.async_copy(u_hbm.at[pl.ds(off0, CH)], pay0, ps0)
        i1 = pltpu.async_copy(dst_hbm.at[pl.ds(off1, CH)], idx1, is1)
        p1 = pltpu.async_copy(u_hbm.at[pl.ds(off1, CH)], pay1, ps1)
        i0.wait()
        p0.wait()
        pltpu.sync_copy(pay0, acc_sh.at[idx0], add=True)
        i1.wait()
        p1.wait()
        pltpu.sync_copy(pay1, acc_sh.at[idx1], add=True)
        return carry

    lax.fori_loop(0, NK // 2, pair, None)
    plsc.subcore_barrier()
    pltpu.sync_copy(acc_sh.at[pl.ds(sid * 640, 640)],
                    out_hbm.at[cid, pl.ds(sid * 640, 640)])


def _sc_scatter(u, dst):
    fn = pl.kernel(
        _sc_scatter_body,
        out_type=jax.ShapeDtypeStruct((NC, NP, D), _f32),
        mesh=_mesh(),
        scratch_types=[
            pltpu.VMEM((CH,), jnp.int32),
            pltpu.VMEM((CH,), jnp.int32),
            pltpu.VMEM((CH, D), _f32),
            pltpu.VMEM((CH, D), _f32),
            pltpu.VMEM_SHARED((NP, D), _f32),
        ] + [pltpu.SemaphoreType.DMA] * 4,
    )
    return fn(u, dst)


def _sc_deg_body(dst_hbm, out_hbm, idx_v, one_v, zer_v, acc_sh, sem):
    cid = lax.axis_index("c")
    sid = lax.axis_index("s")
    wid = sid * NC + cid

    onev = jnp.ones((16,), _f32)
    zvec = jnp.zeros((16,), _f32)

    def obody(i, carry):
        one_v[i, pl.ds(0, 16)] = onev
        return carry

    lax.fori_loop(0, CH, obody, None)

    def zbody(i, carry):
        zer_v[i, pl.ds(0, 16)] = zvec
        return carry

    lax.fori_loop(0, ZR, zbody, None)
    for j in range(5):
        pltpu.sync_copy(zer_v, acc_sh.at[pl.ds(sid * 625 + j * ZR, ZR)])
    plsc.subcore_barrier()

    nk = (NCHUNK - wid + NW - 1) // NW

    def body(k, carry):
        off = (wid + k * NW) * CH
        pltpu.sync_copy(dst_hbm.at[pl.ds(off, CH)], idx_v)
        pltpu.sync_copy(one_v, acc_sh.at[idx_v], add=True)
        return carry

    lax.fori_loop(0, nk, body, None)
    plsc.subcore_barrier()
    pltpu.sync_copy(acc_sh.at[pl.ds(sid * 625, 625)],
                    out_hbm.at[cid, pl.ds(sid * 625, 625)])


def _sc_deg(dst):
    fn = pl.kernel(
        _sc_deg_body,
        out_type=jax.ShapeDtypeStruct((NC, N, ED), _f32),
        mesh=_mesh(),
        scratch_types=[
            pltpu.VMEM((CH,), jnp.int32),
            pltpu.VMEM((CH, ED), _f32),
            pltpu.VMEM((ZR, ED), _f32),
            pltpu.VMEM_SHARED((N, ED), _f32),
            pltpu.SemaphoreType.DMA,
        ],
        compiler_params=pltpu.CompilerParams(use_tc_tiling_on_sc=False),
    )
    return fn(dst)


def _layer(h, ef, dstp, srcp, dsts, deg,
           eW1, eb1, eW2, eb2, nW1, nb1, nW2, nb2):
    xi, xj = _sc_gather(h, dstp, srcp)
    ef2, u = _tcB(xi, xj, ef,
                  eW1[:D], eW1[D:2 * D], eW1[2 * D:], eb1[None, :],
                  eW2, eb2[None, :],
                  nW1[:D], nW1[D:2 * D], nW1[2 * D:], nb1[None, :])
    agg = _sc_scatter(u, dsts)
    h2 = _tcC(agg, deg, nW2, nb2[None, :])
    return h2, ef2


def kernel(x, edge_index, edge_feat,
           l0_eW1, l0_eb1, l0_eW2, l0_eb2, l0_nW1, l0_nb1, l0_nW2, l0_nb2,
           l1_eW1, l1_eb1, l1_eW2, l1_eb2, l1_nW1, l1_nb1, l1_nW2, l1_nb2):
    src = edge_index[0]
    dst = edge_index[1]
    padz = jnp.zeros((EPAD - E,), jnp.int32)
    dstp = jnp.concatenate([dst, padz])
    srcp = jnp.concatenate([src, padz])
    dsts = jnp.concatenate([dst, padz + N])
    deg = _sc_deg(dst)
    h1, ef1 = _layer(x, edge_feat, dstp, srcp, dsts, deg,
                     l0_eW1, l0_eb1, l0_eW2, l0_eb2,
                     l0_nW1, l0_nb1, l0_nW2, l0_nb2)
    h2, ef2 = _layer(h1, ef1, dstp, srcp, dsts, deg,
                     l1_eW1, l1_eb1, l1_eW2, l1_eb2,
                     l1_nW1, l1_nb1, l1_nW2, l1_nb2)
    return (h2, ef2)


# speedup vs baseline: 1.9425x; 1.9425x over previous
import jax
import jax.numpy as jnp
from jax import lax
from jax.experimental import pallas as pl
from jax.experimental.pallas import tpu as pltpu
from jax.experimental.pallas import tpu_sc as plsc

N = 10000
E = 320000
D = 128
ED = 16
NC = 2
NS = 16
NW = NC * NS
CH = 128
NCHUNK = E // CH
NK = 80
EPAD = NW * NK * CH
NB = 1000
EB = 4000
ZR = 128
NP = 10240

_f32 = jnp.float32


def _tcB_body(xi_ref, xj_ref, ef_ref,
              ew1a_ref, ew1b_ref, ew1c_ref, eb1_ref, ew2_ref, eb2_ref,
              nw1a_ref, nw1b_ref, nw1c_ref, nb1_ref,
              ef2_ref, u_ref):
    xi = xi_ref[...]
    xj = xj_ref[...]
    ef = ef_ref[...]
    pe = (jnp.dot(xi, ew1a_ref[...], preferred_element_type=_f32)
          + jnp.dot(xj, ew1b_ref[...], preferred_element_type=_f32)
          + jnp.dot(ef, ew1c_ref[...], preferred_element_type=_f32)
          + eb1_ref[...])
    t = jnp.maximum(pe, 0.0)
    ef2 = jnp.dot(t, ew2_ref[...], preferred_element_type=_f32) + eb2_ref[...]
    ef2_ref[...] = ef2
    pn = (jnp.dot(xi, nw1a_ref[...], preferred_element_type=_f32)
          + jnp.dot(xj, nw1b_ref[...], preferred_element_type=_f32)
          + jnp.dot(ef2, nw1c_ref[...], preferred_element_type=_f32)
          + nb1_ref[...])
    u_ref[...] = jnp.maximum(pn, 0.0)


def _tcB(xi, xj, ef, eW1a, eW1b, eW1c, eb1, eW2, eb2, nW1a, nW1b, nW1c, nb1):
    grid = E // EB
    full = lambda shape: pl.BlockSpec(shape, lambda i: tuple(0 for _ in shape))
    return pl.pallas_call(
        _tcB_body,
        grid=(grid,),
        in_specs=[
            pl.BlockSpec((EB, D), lambda i: (i, 0)),
            pl.BlockSpec((EB, D), lambda i: (i, 0)),
            pl.BlockSpec((EB, ED), lambda i: (i, 0)),
            full((D, 32)), full((D, 32)), full((ED, 32)), full((1, 32)),
            full((32, ED)), full((1, ED)),
            full((D, D)), full((D, D)), full((ED, D)), full((1, D)),
        ],
        out_specs=[
            pl.BlockSpec((EB, ED), lambda i: (i, 0)),
            pl.BlockSpec((EB, D), lambda i: (i, 0)),
        ],
        out_shape=[
            jax.ShapeDtypeStruct((E, ED), _f32),
            jax.ShapeDtypeStruct((EPAD, D), _f32),
        ],
    )(xi, xj, ef, eW1a, eW1b, eW1c, eb1, eW2, eb2, nW1a, nW1b, nW1c, nb1)


def _tcC_body(agg_ref, deg_ref, w_ref, b_ref, h_ref):
    a = agg_ref[0] + agg_ref[1]
    d = deg_ref[0, :, 0:1] + deg_ref[1, :, 0:1]
    h_ref[...] = jnp.dot(a, w_ref[...], preferred_element_type=_f32) + d * b_ref[...]


def _tcC(agg, deg, nW2, nb2):
    grid = N // NB
    return pl.pallas_call(
        _tcC_body,
        grid=(grid,),
        in_specs=[
            pl.BlockSpec((2, NB, D), lambda i: (0, i, 0)),
            pl.BlockSpec((2, NB, ED), lambda i: (0, i, 0)),
            pl.BlockSpec((D, D), lambda i: (0, 0)),
            pl.BlockSpec((1, D), lambda i: (0, 0)),
        ],
        out_specs=pl.BlockSpec((NB, D), lambda i: (i, 0)),
        out_shape=jax.ShapeDtypeStruct((N, D), _f32),
    )(agg, deg, nW2, nb2)


def _mesh():
    return plsc.VectorSubcoreMesh(core_axis_name="c", subcore_axis_name="s",
                                  num_cores=NC, num_subcores=NS)


def _sc_gather_body(h_hbm, dst_hbm, src_hbm, xi_hbm, xj_hbm,
                    idxd_all, idxs_all, xi0, xi1, xj0, xj1,
                    gd0, gd1, gs0, gs1, wd0, wd1, ws0, ws1):
    cid = lax.axis_index("c")
    sid = lax.axis_index("s")
    wid = sid * NC + cid
    base = wid * (NK * CH)
    pltpu.sync_copy(dst_hbm.at[pl.ds(base, NK * CH)], idxd_all)
    pltpu.sync_copy(src_hbm.at[pl.ds(base, NK * CH)], idxs_all)

    def pair(m, carry):
        k0 = 2 * m
        off0 = base + k0 * CH
        off1 = off0 + CH
        cd0 = pltpu.async_copy(h_hbm.at[idxd_all.at[pl.ds(k0 * CH, CH)]], xi0, gd0)
        cs0 = pltpu.async_copy(h_hbm.at[idxs_all.at[pl.ds(k0 * CH, CH)]], xj0, gs0)
        cd1 = pltpu.async_copy(h_hbm.at[idxd_all.at[pl.ds(k0 * CH + CH, CH)]], xi1, gd1)
        cs1 = pltpu.async_copy(h_hbm.at[idxs_all.at[pl.ds(k0 * CH + CH, CH)]], xj1, gs1)
        cd0.wait()
        w0 = pltpu.async_copy(xi0, xi_hbm.at[pl.ds(off0, CH)], wd0)
        cs0.wait()
        w1 = pltpu.async_copy(xj0, xj_hbm.at[pl.ds(off0, CH)], ws0)
        cd1.wait()
        w2 = pltpu.async_copy(xi1, xi_hbm.at[pl.ds(off1, CH)], wd1)
        cs1.wait()
        w3 = pltpu.async_copy(xj1, xj_hbm.at[pl.ds(off1, CH)], ws1)
        w0.wait()
        w1.wait()
        w2.wait()
        w3.wait()
        return carry

    lax.fori_loop(0, NK // 2, pair, None)


def _sc_gather(h, dst, src):
    fn = pl.kernel(
        _sc_gather_body,
        out_type=[
            jax.ShapeDtypeStruct((EPAD, D), _f32),
            jax.ShapeDtypeStruct((EPAD, D), _f32),
        ],
        mesh=_mesh(),
        scratch_types=[
            pltpu.VMEM((NK * CH,), jnp.int32),
            pltpu.VMEM((NK * CH,), jnp.int32),
            pltpu.VMEM((CH, D), _f32),
            pltpu.VMEM((CH, D), _f32),
            pltpu.VMEM((CH, D), _f32),
            pltpu.VMEM((CH, D), _f32),
        ] + [pltpu.SemaphoreType.DMA] * 8,
    )
    return fn(h, dst, src)


def _sc_scatter_body(u_hbm, dst_hbm, out_hbm, idx0, idx1, pay0, pay1, acc_sh,
                     is0, is1, ps0, ps1):
    cid = lax.axis_index("c")
    sid = lax.axis_index("s")
    wid = sid * NC + cid

    zvec = jnp.zeros((16,), _f32)

    def zbody(i, carry):
        r = i // (D // 16)
        c = (i % (D // 16)) * 16
        pay0[r, pl.ds(c, 16)] = zvec
        return carry

    lax.fori_loop(0, CH * (D // 16), zbody, None)
    for j in range(5):
        pltpu.sync_copy(pay0, acc_sh.at[pl.ds(sid * 640 + j * CH, CH)])
    plsc.subcore_barrier()

    base = wid * (NK * CH)

    def pair(m, carry):
        off0 = base + (2 * m) * CH
        off1 = off0 + CH
        i0 = pltpu.async_copy(dst_hbm.at[pl.ds(off0, CH)], idx0, is0)
        p0 = pltpu.async_copy(u_hbm.at[pl.ds(off0, CH)], pay0, ps0)
        i1 = pltpu.async_copy(dst_hbm.at[pl.ds(off1, CH)], idx1, is1)
        p1 = pltpu.async_copy(u_hbm.at[pl.ds(off1, CH)], pay1, ps1)
        i0.wait()
        p0.wait()
        pltpu.sync_copy(pay0, acc_sh.at[idx0], add=True)
        i1.wait()
        p1.wait()
        pltpu.sync_copy(pay1, acc_sh.at[idx1], add=True)
        return carry

    lax.fori_loop(0, NK // 2, pair, None)
    plsc.subcore_barrier()
    pltpu.sync_copy(acc_sh.at[pl.ds(sid * 640, 640)],
                    out_hbm.at[cid, pl.ds(sid * 640, 640)])


def _sc_scatter(u, dst):
    fn = pl.kernel(
        _sc_scatter_body,
        out_type=jax.ShapeDtypeStruct((NC, NP, D), _f32),
        mesh=_mesh(),
        scratch_types=[
            pltpu.VMEM((CH,), jnp.int32),
            pltpu.VMEM((CH,), jnp.int32),
            pltpu.VMEM((CH, D), _f32),
            pltpu.VMEM((CH, D), _f32),
            pltpu.VMEM_SHARED((NP, D), _f32),
        ] + [pltpu.SemaphoreType.DMA] * 4,
    )
    return fn(u, dst)


def _sc_deg_body(dst_hbm, out_hbm, idx_v, one_v, zer_v, acc_sh, sem):
    cid = lax.axis_index("c")
    sid = lax.axis_index("s")
    wid = sid * NC + cid

    onev = jnp.ones((16,), _f32)
    zvec = jnp.zeros((16,), _f32)

    def obody(i, carry):
        one_v[i, pl.ds(0, 16)] = onev
        return carry

    lax.fori_loop(0, CH, obody, None)

    def zbody(i, carry):
        zer_v[i, pl.ds(0, 16)] = zvec
        return carry

    lax.fori_loop(0, ZR, zbody, None)
    for j in range(5):
        pltpu.sync_copy(zer_v, acc_sh.at[pl.ds(sid * 625 + j * ZR, ZR)])
    plsc.subcore_barrier()

    nk = (NCHUNK - wid + NW - 1) // NW

    def body(k, carry):
        off = (wid + k * NW) * CH
        pltpu.sync_copy(dst_hbm.at[pl.ds(off, CH)], idx_v)
        pltpu.sync_copy(one_v, acc_sh.at[idx_v], add=True)
        return carry

    lax.fori_loop(0, nk, body, None)
    plsc.subcore_barrier()
    pltpu.sync_copy(acc_sh.at[pl.ds(sid * 625, 625)],
                    out_hbm.at[cid, pl.ds(sid * 625, 625)])


def _sc_deg(dst):
    fn = pl.kernel(
        _sc_deg_body,
        out_type=jax.ShapeDtypeStruct((NC, N, ED), _f32),
        mesh=_mesh(),
        scratch_types=[
            pltpu.VMEM((CH,), jnp.int32),
            pltpu.VMEM((CH, ED), _f32),
            pltpu.VMEM((ZR, ED), _f32),
            pltpu.VMEM_SHARED((N, ED), _f32),
            pltpu.SemaphoreType.DMA,
        ],
        compiler_params=pltpu.CompilerParams(use_tc_tiling_on_sc=False),
    )
    return fn(dst)


def _layer(h, ef, dstp, srcp, dsts, deg,
           eW1, eb1, eW2, eb2, nW1, nb1, nW2, nb2):
    xi, xj = _sc_gather(h, dstp, srcp)
    ef2, u = _tcB(xi, xj, ef,
                  eW1[:D], eW1[D:2 * D], eW1[2 * D:], eb1[None, :],
                  eW2, eb2[None, :],
                  nW1[:D], nW1[D:2 * D], nW1[2 * D:], nb1[None, :])
    agg = _sc_scatter(u, dsts)
    h2 = _tcC(agg, deg, nW2, nb2[None, :])
    return h2, ef2


def kernel(x, edge_index, edge_feat,
           l0_eW1, l0_eb1, l0_eW2, l0_eb2, l0_nW1, l0_nb1, l0_nW2, l0_nb2,
           l1_eW1, l1_eb1, l1_eW2, l1_eb2, l1_nW1, l1_nb1, l1_nW2, l1_nb2):
    src = edge_index[0]
    dst = edge_index[1]
    padr = jnp.arange(EPAD - E, dtype=jnp.int32)
    dstp = jnp.concatenate([dst, padr % N])
    srcp = jnp.concatenate([src, (padr * 7 + 3) % N])
    dsts = jnp.concatenate([dst, N + padr % (NP - N)])
    deg = _sc_deg(dst)
    h1, ef1 = _layer(x, edge_feat, dstp, srcp, dsts, deg,
                     l0_eW1, l0_eb1, l0_eW2, l0_eb2,
                     l0_nW1, l0_nb1, l0_nW2, l0_nb2)
    h2, ef2 = _layer(h1, ef1, dstp, srcp, dsts, deg,
                     l1_eW1, l1_eb1, l1_eW2, l1_eb2,
                     l1_nW1, l1_nb1, l1_nW2, l1_nb2)
    return (h2, ef2)


# speedup vs baseline: 2.0035x; 1.0314x over previous
import jax
import jax.numpy as jnp
from jax import lax
from jax.experimental import pallas as pl
from jax.experimental.pallas import tpu as pltpu
from jax.experimental.pallas import tpu_sc as plsc

N = 10000
E = 320000
D = 128
ED = 16
NC = 2
NS = 16
NW = NC * NS
CH = 128
NCHUNK = E // CH
NK = 80
EPAD = NW * NK * CH
NB = 1000
EB = 8000
ZR = 128
NP = 10240

_f32 = jnp.float32


def _tcB_body(xi_ref, xj_ref, ef_ref,
              ew1a_ref, ew1b_ref, ew1c_ref, eb1_ref, ew2_ref, eb2_ref,
              nw1a_ref, nw1b_ref, nw1c_ref, nb1_ref,
              ef2_ref, u_ref):
    xi = xi_ref[...]
    xj = xj_ref[...]
    ef = ef_ref[...]
    pe = (jnp.dot(xi, ew1a_ref[...], preferred_element_type=_f32)
          + jnp.dot(xj, ew1b_ref[...], preferred_element_type=_f32)
          + jnp.dot(ef, ew1c_ref[...], preferred_element_type=_f32)
          + eb1_ref[...])
    t = jnp.maximum(pe, 0.0)
    ef2 = jnp.dot(t, ew2_ref[...], preferred_element_type=_f32) + eb2_ref[...]
    ef2_ref[...] = ef2
    pn = (jnp.dot(xi, nw1a_ref[...], preferred_element_type=_f32)
          + jnp.dot(xj, nw1b_ref[...], preferred_element_type=_f32)
          + jnp.dot(ef2, nw1c_ref[...], preferred_element_type=_f32)
          + nb1_ref[...])
    u_ref[...] = jnp.maximum(pn, 0.0)


def _tcB(xi, xj, ef, eW1a, eW1b, eW1c, eb1, eW2, eb2, nW1a, nW1b, nW1c, nb1):
    grid = E // EB
    full = lambda shape: pl.BlockSpec(shape, lambda i: tuple(0 for _ in shape))
    return pl.pallas_call(
        _tcB_body,
        grid=(grid,),
        in_specs=[
            pl.BlockSpec((EB, D), lambda i: (i, 0)),
            pl.BlockSpec((EB, D), lambda i: (i, 0)),
            pl.BlockSpec((EB, ED), lambda i: (i, 0)),
            full((D, 32)), full((D, 32)), full((ED, 32)), full((1, 32)),
            full((32, ED)), full((1, ED)),
            full((D, D)), full((D, D)), full((ED, D)), full((1, D)),
        ],
        out_specs=[
            pl.BlockSpec((EB, ED), lambda i: (i, 0)),
            pl.BlockSpec((EB, D), lambda i: (i, 0)),
        ],
        out_shape=[
            jax.ShapeDtypeStruct((E, ED), _f32),
            jax.ShapeDtypeStruct((EPAD, D), _f32),
        ],
    )(xi, xj, ef, eW1a, eW1b, eW1c, eb1, eW2, eb2, nW1a, nW1b, nW1c, nb1)


def _tcC_body(agg_ref, deg_ref, w_ref, b_ref, h_ref):
    a = agg_ref[0] + agg_ref[1]
    d = deg_ref[0, :, 0:1] + deg_ref[1, :, 0:1]
    h_ref[...] = jnp.dot(a, w_ref[...], preferred_element_type=_f32) + d * b_ref[...]


def _tcC(agg, deg, nW2, nb2):
    grid = N // NB
    return pl.pallas_call(
        _tcC_body,
        grid=(grid,),
        in_specs=[
            pl.BlockSpec((2, NB, D), lambda i: (0, i, 0)),
            pl.BlockSpec((2, NB, ED), lambda i: (0, i, 0)),
            pl.BlockSpec((D, D), lambda i: (0, 0)),
            pl.BlockSpec((1, D), lambda i: (0, 0)),
        ],
        out_specs=pl.BlockSpec((NB, D), lambda i: (i, 0)),
        out_shape=jax.ShapeDtypeStruct((N, D), _f32),
    )(agg, deg, nW2, nb2)


def _mesh():
    return plsc.VectorSubcoreMesh(core_axis_name="c", subcore_axis_name="s",
                                  num_cores=NC, num_subcores=NS)


def _sc_gather_body(h_hbm, dst_hbm, src_hbm, xi_hbm, xj_hbm,
                    idxd_all, idxs_all, xi0, xi1, xj0, xj1,
                    gd0, gd1, gs0, gs1, wd0, wd1, ws0, ws1):
    cid = lax.axis_index("c")
    sid = lax.axis_index("s")
    wid = sid * NC + cid
    base = wid * (NK * CH)
    pltpu.sync_copy(dst_hbm.at[pl.ds(base, NK * CH)], idxd_all)
    pltpu.sync_copy(src_hbm.at[pl.ds(base, NK * CH)], idxs_all)

    def pair(m, carry):
        k0 = 2 * m
        off0 = base + k0 * CH
        off1 = off0 + CH
        cd0 = pltpu.async_copy(h_hbm.at[idxd_all.at[pl.ds(k0 * CH, CH)]], xi0, gd0)
        cs0 = pltpu.async_copy(h_hbm.at[idxs_all.at[pl.ds(k0 * CH, CH)]], xj0, gs0)
        cd1 = pltpu.async_copy(h_hbm.at[idxd_all.at[pl.ds(k0 * CH + CH, CH)]], xi1, gd1)
        cs1 = pltpu.async_copy(h_hbm.at[idxs_all.at[pl.ds(k0 * CH + CH, CH)]], xj1, gs1)
        cd0.wait()
        w0 = pltpu.async_copy(xi0, xi_hbm.at[pl.ds(off0, CH)], wd0)
        cs0.wait()
        w1 = pltpu.async_copy(xj0, xj_hbm.at[pl.ds(off0, CH)], ws0)
        cd1.wait()
        w2 = pltpu.async_copy(xi1, xi_hbm.at[pl.ds(off1, CH)], wd1)
        cs1.wait()
        w3 = pltpu.async_copy(xj1, xj_hbm.at[pl.ds(off1, CH)], ws1)
        w0.wait()
        w1.wait()
        w2.wait()
        w3.wait()
        return carry

    lax.fori_loop(0, NK // 2, pair, None)


def _sc_gather(h, dst, src):
    fn = pl.kernel(
        _sc_gather_body,
        out_type=[
            jax.ShapeDtypeStruct((EPAD, D), _f32),
            jax.ShapeDtypeStruct((EPAD, D), _f32),
        ],
        mesh=_mesh(),
        scratch_types=[
            pltpu.VMEM((NK * CH,), jnp.int32),
            pltpu.VMEM((NK * CH,), jnp.int32),
            pltpu.VMEM((CH, D), _f32),
            pltpu.VMEM((CH, D), _f32),
            pltpu.VMEM((CH, D), _f32),
            pltpu.VMEM((CH, D), _f32),
        ] + [pltpu.SemaphoreType.DMA] * 8,
    )
    return fn(h, dst, src)


def _sc_scatter_body(u_hbm, dst_hbm, out_hbm, idx0, idx1, pay0, pay1, acc_sh,
                     is0, is1, ps0, ps1):
    cid = lax.axis_index("c")
    sid = lax.axis_index("s")
    wid = sid * NC + cid

    zvec = jnp.zeros((16,), _f32)

    def zbody(i, carry):
        r = i // (D // 16)
        c = (i % (D // 16)) * 16
        pay0[r, pl.ds(c, 16)] = zvec
        return carry

    lax.fori_loop(0, CH * (D // 16), zbody, None)
    for j in range(5):
        pltpu.sync_copy(pay0, acc_sh.at[pl.ds(sid * 640 + j * CH, CH)])
    plsc.subcore_barrier()

    base = wid * (NK * CH)

    def pair(m, carry):
        off0 = base + (2 * m) * CH
        off1 = off0 + CH
        i0 = pltpu.async_copy(dst_hbm.at[pl.ds(off0, CH)], idx0, is0)
        p0 = pltpu.async_copy(u_hbm.at[pl.ds(off0, CH)], pay0, ps0)
        i1 = pltpu.async_copy(dst_hbm.at[pl.ds(off1, CH)], idx1, is1)
        p1 = pltpu.async_copy(u_hbm.at[pl.ds(off1, CH)], pay1, ps1)
        i0.wait()
        p0.wait()
        pltpu.sync_copy(pay0, acc_sh.at[idx0], add=True)
        i1.wait()
        p1.wait()
        pltpu.sync_copy(pay1, acc_sh.at[idx1], add=True)
        return carry

    lax.fori_loop(0, NK // 2, pair, None)
    plsc.subcore_barrier()
    pltpu.sync_copy(acc_sh.at[pl.ds(sid * 640, 640)],
                    out_hbm.at[cid, pl.ds(sid * 640, 640)])


def _sc_scatter(u, dst):
    fn = pl.kernel(
        _sc_scatter_body,
        out_type=jax.ShapeDtypeStruct((NC, NP, D), _f32),
        mesh=_mesh(),
        scratch_types=[
            pltpu.VMEM((CH,), jnp.int32),
            pltpu.VMEM((CH,), jnp.int32),
            pltpu.VMEM((CH, D), _f32),
            pltpu.VMEM((CH, D), _f32),
            pltpu.VMEM_SHARED((NP, D), _f32),
        ] + [pltpu.SemaphoreType.DMA] * 4,
    )
    return fn(u, dst)


def _sc_deg_body(dst_hbm, out_hbm, idx_v, one_v, zer_v, acc_sh, sem):
    cid = lax.axis_index("c")
    sid = lax.axis_index("s")
    wid = sid * NC + cid

    onev = jnp.ones((16,), _f32)
    zvec = jnp.zeros((16,), _f32)

    def obody(i, carry):
        one_v[i, pl.ds(0, 16)] = onev
        return carry

    lax.fori_loop(0, CH, obody, None)

    def zbody(i, carry):
        zer_v[i, pl.ds(0, 16)] = zvec
        return carry

    lax.fori_loop(0, ZR, zbody, None)
    for j in range(5):
        pltpu.sync_copy(zer_v, acc_sh.at[pl.ds(sid * 625 + j * ZR, ZR)])
    plsc.subcore_barrier()

    nk = (NCHUNK - wid + NW - 1) // NW

    def body(k, carry):
        off = (wid + k * NW) * CH
        pltpu.sync_copy(dst_hbm.at[pl.ds(off, CH)], idx_v)
        pltpu.sync_copy(one_v, acc_sh.at[idx_v], add=True)
        return carry

    lax.fori_loop(0, nk, body, None)
    plsc.subcore_barrier()
    pltpu.sync_copy(acc_sh.at[pl.ds(sid * 625, 625)],
                    out_hbm.at[cid, pl.ds(sid * 625, 625)])


def _sc_deg(dst):
    fn = pl.kernel(
        _sc_deg_body,
        out_type=jax.ShapeDtypeStruct((NC, N, ED), _f32),
        mesh=_mesh(),
        scratch_types=[
            pltpu.VMEM((CH,), jnp.int32),
            pltpu.VMEM((CH, ED), _f32),
            pltpu.VMEM((ZR, ED), _f32),
            pltpu.VMEM_SHARED((N, ED), _f32),
            pltpu.SemaphoreType.DMA,
        ],
        compiler_params=pltpu.CompilerParams(use_tc_tiling_on_sc=False),
    )
    return fn(dst)


def _layer(h, ef, dstp, srcp, dsts, deg,
           eW1, eb1, eW2, eb2, nW1, nb1, nW2, nb2):
    xi, xj = _sc_gather(h, dstp, srcp)
    ef2, u = _tcB(xi, xj, ef,
                  eW1[:D], eW1[D:2 * D], eW1[2 * D:], eb1[None, :],
                  eW2, eb2[None, :],
                  nW1[:D], nW1[D:2 * D], nW1[2 * D:], nb1[None, :])
    agg = _sc_scatter(u, dsts)
    h2 = _tcC(agg, deg, nW2, nb2[None, :])
    return h2, ef2


def kernel(x, edge_index, edge_feat,
           l0_eW1, l0_eb1, l0_eW2, l0_eb2, l0_nW1, l0_nb1, l0_nW2, l0_nb2,
           l1_eW1, l1_eb1, l1_eW2, l1_eb2, l1_nW1, l1_nb1, l1_nW2, l1_nb2):
    src = edge_index[0]
    dst = edge_index[1]
    padr = jnp.arange(EPAD - E, dtype=jnp.int32)
    dstp = jnp.concatenate([dst, padr % N])
    srcp = jnp.concatenate([src, (padr * 7 + 3) % N])
    dsts = jnp.concatenate([dst, N + padr % (NP - N)])
    deg = _sc_deg(dst)
    h1, ef1 = _layer(x, edge_feat, dstp, srcp, dsts, deg,
                     l0_eW1, l0_eb1, l0_eW2, l0_eb2,
                     l0_nW1, l0_nb1, l0_nW2, l0_nb2)
    h2, ef2 = _layer(h1, ef1, dstp, srcp, dsts, deg,
                     l1_eW1, l1_eb1, l1_eW2, l1_eb2,
                     l1_nW1, l1_nb1, l1_nW2, l1_nb2)
    return (h2, ef2)
